# baseline (speedup 1.0000x reference)
import jax
import jax.numpy as jnp
from jax import lax
from jax.experimental import pallas as pl
from jax.experimental.pallas import tpu as pltpu
from jax.experimental.pallas import tpu_sc as plsc

NC = 2
NS = 16
NW = NC * NS
CH = 128


def _mesh():
    return plsc.VectorSubcoreMesh(
        core_axis_name="c", subcore_axis_name="s", num_cores=NC, num_subcores=NS
    )


def _make_deg_kernel(n_pad, chunks):
    cpw = chunks // NW
    rpt = n_pad // NS
    lag = 8

    def body(dsts, e0, zeros, out, didx_all, e0_v, acc, sem):
        c = lax.axis_index("c")
        s = lax.axis_index("s")
        wid = s * NC + c
        pltpu.sync_copy(dsts.at[pl.ds(wid * cpw, cpw)], didx_all)
        pltpu.sync_copy(e0, e0_v)
        pltpu.sync_copy(zeros.at[pl.ds(s * rpt, rpt)], acc.at[pl.ds(s * rpt, rpt)])
        plsc.subcore_barrier()

        for j in range(cpw):
            pltpu.async_copy(e0_v, acc.at[didx_all.at[j]], sem, add=True)
            if j >= lag:
                pltpu.make_async_copy(e0, e0_v, sem).wait()
        for _ in range(min(lag, cpw)):
            pltpu.make_async_copy(e0, e0_v, sem).wait()

        plsc.subcore_barrier()
        pltpu.sync_copy(acc.at[pl.ds(s * rpt, rpt)], out.at[c, pl.ds(s * rpt, rpt)])

    return pl.kernel(
        body,
        out_type=jax.ShapeDtypeStruct((NC, n_pad, 16), jnp.float32),
        mesh=_mesh(),
        scratch_types=[
            pltpu.VMEM((cpw, CH), jnp.int32),
            pltpu.VMEM((CH, 16), jnp.float32),
            pltpu.VMEM_SHARED((n_pad, 16), jnp.float32),
            pltpu.SemaphoreType.DMA,
        ],
        compiler_params=pltpu.CompilerParams(use_tc_tiling_on_sc=False),
    )


def _make_agg_kernel(n_pad, chunks, d):
    cpw = chunks // NW
    rpt = n_pad // NS
    k = 8
    ngroups = cpw // k
    assert ngroups % 2 == 0 and ngroups >= 2

    def body(srcs, dsts, y, zeros, out, sidx_all, didx_all, r0, r1, acc,
             gs0, gs1, ss0, ss1):
        c = lax.axis_index("c")
        s = lax.axis_index("s")
        wid = s * NC + c
        base = wid * cpw
        pltpu.sync_copy(srcs.at[pl.ds(base, cpw)], sidx_all)
        pltpu.sync_copy(dsts.at[pl.ds(base, cpw)], didx_all)
        pltpu.sync_copy(zeros.at[pl.ds(s * rpt, rpt)], acc.at[pl.ds(s * rpt, rpt)])
        plsc.subcore_barrier()

        rows = (r0, r1)
        gsems = (gs0, gs1)
        ssems = (ss0, ss1)

        def start_gathers(g, sk):
            for b in range(k):
                pltpu.async_copy(y.at[sidx_all.at[g * k + b]], rows[sk].at[b],
                                 gsems[sk])

        def wait_gathers(sk):
            for b in range(k):
                pltpu.make_async_copy(y.at[pl.ds(0, CH)], rows[sk].at[b],
                                      gsems[sk]).wait()

        def start_scatters(g, sk):
            for b in range(k):
                pltpu.async_copy(rows[sk].at[b], acc.at[didx_all.at[g * k + b]],
                                 ssems[sk], add=True)

        def wait_scatters(sk):
            for b in range(k):
                pltpu.make_async_copy(y.at[pl.ds(0, CH)], rows[sk].at[b],
                                      ssems[sk]).wait()

        start_gathers(0, 0)
        for g in range(ngroups):
            cur = g % 2
            nxt = 1 - cur
            wait_gathers(cur)
            if g + 1 < ngroups:
                if g >= 1:
                    wait_scatters(nxt)
                start_gathers(g + 1, nxt)
            start_scatters(g, cur)
        wait_scatters((ngroups - 2) % 2)
        wait_scatters((ngroups - 1) % 2)

        plsc.subcore_barrier()
        pltpu.sync_copy(acc.at[pl.ds(s * rpt, rpt)], out.at[c, pl.ds(s * rpt, rpt)])

    return pl.kernel(
        body,
        out_type=jax.ShapeDtypeStruct((NC, n_pad, d), jnp.float32),
        mesh=_mesh(),
        scratch_types=[
            pltpu.VMEM((cpw, CH), jnp.int32),
            pltpu.VMEM((cpw, CH), jnp.int32),
            pltpu.VMEM((k, CH, d), jnp.float32),
            pltpu.VMEM((k, CH, d), jnp.float32),
            pltpu.VMEM_SHARED((n_pad, d), jnp.float32),
            pltpu.SemaphoreType.DMA,
            pltpu.SemaphoreType.DMA,
            pltpu.SemaphoreType.DMA,
            pltpu.SemaphoreType.DMA,
        ],
        compiler_params=pltpu.CompilerParams(use_tc_tiling_on_sc=(d % 128 == 0)),
    )


def _make_agg_kernel_wide(n_pad, chunks, d):
    cpw = chunks // NW
    rpt = n_pad // NS
    assert cpw >= 2

    def body(srcs, dsts, y, zeros, out, s0, s1, d0, d1, r0, r1, acc,
             gs0, gs1, ss0, ss1):
        c = lax.axis_index("c")
        s = lax.axis_index("s")
        wid = s * NC + c
        base = wid * cpw
        pltpu.sync_copy(zeros.at[pl.ds(s * rpt, rpt)], acc.at[pl.ds(s * rpt, rpt)])
        plsc.subcore_barrier()

        sidx = (s0, s1)
        didx = (d0, d1)
        rows = (r0, r1)
        gsems = (gs0, gs1)
        ssems = (ss0, ss1)

        def load_idx(j, sk):
            pltpu.sync_copy(srcs.at[base + j], sidx[sk])
            pltpu.sync_copy(dsts.at[base + j], didx[sk])

        def start_gather(sk):
            pltpu.async_copy(y.at[sidx[sk]], rows[sk], gsems[sk])

        def wait_gather(sk):
            pltpu.make_async_copy(y.at[pl.ds(0, CH)], rows[sk], gsems[sk]).wait()

        def start_scatter(sk):
            pltpu.async_copy(rows[sk], acc.at[didx[sk]], ssems[sk], add=True)

        def wait_scatter(sk):
            pltpu.make_async_copy(y.at[pl.ds(0, CH)], rows[sk], ssems[sk]).wait()

        load_idx(0, 0)
        start_gather(0)
        for j in range(cpw):
            cur = j % 2
            nxt = 1 - cur
            if j + 1 < cpw:
                if j >= 1:
                    wait_scatter(nxt)
                load_idx(j + 1, nxt)
                start_gather(nxt)
            wait_gather(cur)
            start_scatter(cur)
        wait_scatter(0)
        wait_scatter(1)

        plsc.subcore_barrier()
        pltpu.sync_copy(acc.at[pl.ds(s * rpt, rpt)], out.at[c, pl.ds(s * rpt, rpt)])

    return pl.kernel(
        body,
        out_type=jax.ShapeDtypeStruct((NC, n_pad, d), jnp.float32),
        mesh=_mesh(),
        scratch_types=[
            pltpu.VMEM((CH,), jnp.int32),
            pltpu.VMEM((CH,), jnp.int32),
            pltpu.VMEM((CH,), jnp.int32),
            pltpu.VMEM((CH,), jnp.int32),
            pltpu.VMEM((CH, d), jnp.float32),
            pltpu.VMEM((CH, d), jnp.float32),
            pltpu.VMEM_SHARED((n_pad, d), jnp.float32),
            pltpu.SemaphoreType.DMA,
            pltpu.SemaphoreType.DMA,
            pltpu.SemaphoreType.DMA,
            pltpu.SemaphoreType.DMA,
        ],
        compiler_params=pltpu.CompilerParams(use_tc_tiling_on_sc=(d % 128 == 0)),
    )


_DOT = dict(preferred_element_type=jnp.float32, precision=lax.Precision.HIGHEST)


def _tc_pre(degout, x, w1, r):
    n, f = x.shape

    def body(deg_ref, x_ref, w_ref, y_ref, dis_ref):
        cnt = deg_ref[0, :, 0:1] + deg_ref[1, :, 0:1]
        dis = lax.rsqrt(cnt + 1.0)
        y_ref[...] = dis * jnp.dot(x_ref[...], w_ref[...], **_DOT)
        dis_ref[...] = dis

    return pl.pallas_call(
        body,
        grid=(n // r,),
        in_specs=[
            pl.BlockSpec((NC, r, 16), lambda i: (0, i, 0)),
            pl.BlockSpec((r, f), lambda i: (i, 0)),
            pl.BlockSpec((f, f), lambda i: (0, 0)),
        ],
        out_specs=[
            pl.BlockSpec((r, f), lambda i: (i, 0)),
            pl.BlockSpec((r, 1), lambda i: (i, 0)),
        ],
        out_shape=[
            jax.ShapeDtypeStruct((n, f), jnp.float32),
            jax.ShapeDtypeStruct((n, 1), jnp.float32),
        ],
    )(degout, x, w1)


def _tc_mid(agg1, y1, dis, b1, w2, r):
    n, f = y1.shape
    f2 = w2.shape[1]

    def body(a_ref, y1_ref, dis_ref, b_ref, w_ref, y2_ref):
        a = a_ref[0] + a_ref[1]
        dis = dis_ref[...]
        h = jnp.maximum(dis * (a + y1_ref[...]) + b_ref[...], 0.0)
        y2_ref[...] = dis * jnp.dot(h, w_ref[...], **_DOT)

    return pl.pallas_call(
        body,
        grid=(n // r,),
        in_specs=[
            pl.BlockSpec((NC, r, f), lambda i: (0, i, 0)),
            pl.BlockSpec((r, f), lambda i: (i, 0)),
            pl.BlockSpec((r, 1), lambda i: (i, 0)),
            pl.BlockSpec((1, f), lambda i: (0, 0)),
            pl.BlockSpec((f, f2), lambda i: (0, 0)),
        ],
        out_specs=pl.BlockSpec((r, f2), lambda i: (i, 0)),
        out_shape=jax.ShapeDtypeStruct((n, f2), jnp.float32),
    )(agg1, y1, dis, b1, w2)


def _tc_post(agg2, y2, dis, b2, fcw_pad, fcb_pad, r):
    n, f2 = y2.shape
    w = fcw_pad.shape[1]

    def body(a_ref, y2_ref, dis_ref, b_ref, fw_ref, fb_ref, o_ref):
        a = a_ref[0] + a_ref[1]
        dis = dis_ref[...]
        h = jnp.maximum(dis * (a + y2_ref[...]) + b_ref[...], 0.0)
        lp = jnp.dot(h, fw_ref[...], **_DOT) + fb_ref[...]
        m = jnp.max(lp, axis=1, keepdims=True)
        ssum = jnp.sum(jnp.exp(lp - m), axis=1, keepdims=True)
        o_ref[...] = lp - m - jnp.log(ssum)

    return pl.pallas_call(
        body,
        grid=(n // r,),
        in_specs=[
            pl.BlockSpec((NC, r, f2), lambda i: (0, i, 0)),
            pl.BlockSpec((r, f2), lambda i: (i, 0)),
            pl.BlockSpec((r, 1), lambda i: (i, 0)),
            pl.BlockSpec((1, f2), lambda i: (0, 0)),
            pl.BlockSpec((f2, w), lambda i: (0, 0)),
            pl.BlockSpec((1, w), lambda i: (0, 0)),
        ],
        out_specs=pl.BlockSpec((r, w), lambda i: (i, 0)),
        out_shape=jax.ShapeDtypeStruct((n, w), jnp.float32),
    )(agg2, y2, dis, b2, fcw_pad, fcb_pad)


def kernel(x, edge_index, W1, b1, W2, b2, fc_W, fc_b):
    n, f = x.shape
    e = edge_index.shape[1]
    f2 = W2.shape[1]
    ncls = fc_W.shape[1]
    r = 2000

    src = edge_index[0].astype(jnp.int32)
    dst = edge_index[1].astype(jnp.int32)
    cpw = -(-(-(-e // (NW * CH))) // 16) * 16
    e_pad = NW * CH * cpw
    chunks = e_pad // CH
    pad = e_pad - e
    srcs = jnp.concatenate([src, jnp.zeros((pad,), jnp.int32)]).reshape(chunks, CH)
    dsts = jnp.concatenate([dst, jnp.full((pad,), n, jnp.int32)]).reshape(chunks, CH)

    n_pad = -(-(n + 1) // (NS * 8)) * (NS * 8)
    z16 = jnp.zeros((n_pad, 16), jnp.float32)
    zf = jnp.zeros((n_pad, f), jnp.float32)
    e0 = jnp.zeros((CH, 16), jnp.float32).at[:, 0].set(1.0)

    degout = _make_deg_kernel(n_pad, chunks)(dsts, e0, z16)
    y1, dis = _tc_pre(degout, x, W1, r)
    agg1 = _make_agg_kernel_wide(n_pad, chunks, f)(srcs, dsts, y1, zf)
    y2 = _tc_mid(agg1, y1, dis, b1.reshape(1, f), W2, r)
    agg2 = _make_agg_kernel(n_pad, chunks, f2)(srcs, dsts, y2, z16[:, :f2])
    fcw_pad = jnp.zeros((f2, 128), jnp.float32).at[:, :ncls].set(fc_W)
    fcb_pad = jnp.full((1, 128), -1e30, jnp.float32).at[0, :ncls].set(fc_b)
    out = _tc_post(agg2, y2, dis, b2.reshape(1, f2), fcw_pad, fcb_pad, r)
    return out[:, :ncls]


# speedup vs baseline: 1.0708x; 1.0708x over previous
import jax
import jax.numpy as jnp
from jax import lax
from jax.experimental import pallas as pl
from jax.experimental.pallas import tpu as pltpu
from jax.experimental.pallas import tpu_sc as plsc

NC = 2
NS = 16
NW = NC * NS
CH = 128

SHARE0_DEG = 0.6
SHARE0_AGG16 = 0.7
SHARE0_AGG128 = 0.75


def _mesh():
    return plsc.VectorSubcoreMesh(
        core_axis_name="c", subcore_axis_name="s", num_cores=NC, num_subcores=NS
    )


def _core_split(chunks, share0, gran):
    per = chunks // NS
    cpw0 = min(per - gran, max(gran, int(per * share0 // gran) * gran))
    return cpw0, per - cpw0


def _make_deg_kernel(n_pad, chunks):
    cpw0, cpw1 = _core_split(chunks, SHARE0_DEG, 8)
    rpt = n_pad // NS
    lag = 8

    def body(dsts, e0, zeros, out, didx_all, e0_v, acc, sem):
        c = lax.axis_index("c")
        s = lax.axis_index("s")
        pltpu.sync_copy(e0, e0_v)
        pltpu.sync_copy(zeros.at[pl.ds(s * rpt, rpt)], acc.at[pl.ds(s * rpt, rpt)])
        plsc.subcore_barrier()

        def run(cpw, start):
            base = start + s * cpw
            pltpu.sync_copy(dsts.at[pl.ds(base, cpw)], didx_all.at[pl.ds(0, cpw)])
            for j in range(cpw):
                pltpu.async_copy(e0_v, acc.at[didx_all.at[j]], sem, add=True)
                if j >= lag:
                    pltpu.make_async_copy(e0, e0_v, sem).wait()
            for _ in range(min(lag, cpw)):
                pltpu.make_async_copy(e0, e0_v, sem).wait()

        @pl.when(c == 0)
        def _():
            run(cpw0, 0)

        @pl.when(c == 1)
        def _():
            run(cpw1, NS * cpw0)

        plsc.subcore_barrier()
        pltpu.sync_copy(acc.at[pl.ds(s * rpt, rpt)], out.at[c, pl.ds(s * rpt, rpt)])

    return pl.kernel(
        body,
        out_type=jax.ShapeDtypeStruct((NC, n_pad, 16), jnp.float32),
        mesh=_mesh(),
        scratch_types=[
            pltpu.VMEM((max(cpw0, cpw1), CH), jnp.int32),
            pltpu.VMEM((CH, 16), jnp.float32),
            pltpu.VMEM_SHARED((n_pad, 16), jnp.float32),
            pltpu.SemaphoreType.DMA,
        ],
        compiler_params=pltpu.CompilerParams(use_tc_tiling_on_sc=False),
    )


def _make_agg_kernel(n_pad, chunks, d):
    k = 8
    cpw0, cpw1 = _core_split(chunks, SHARE0_AGG16, 2 * k)
    rpt = n_pad // NS

    def body(srcs, dsts, y, zeros, out, sidx_all, didx_all, r0, r1, acc,
             gs0, gs1, ss0, ss1):
        c = lax.axis_index("c")
        s = lax.axis_index("s")
        pltpu.sync_copy(zeros.at[pl.ds(s * rpt, rpt)], acc.at[pl.ds(s * rpt, rpt)])
        plsc.subcore_barrier()

        rows = (r0, r1)
        gsems = (gs0, gs1)
        ssems = (ss0, ss1)

        def run(cpw, start):
            ngroups = cpw // k
            base = start + s * cpw
            pltpu.sync_copy(srcs.at[pl.ds(base, cpw)], sidx_all.at[pl.ds(0, cpw)])
            pltpu.sync_copy(dsts.at[pl.ds(base, cpw)], didx_all.at[pl.ds(0, cpw)])

            def start_gathers(g, sk):
                for b in range(k):
                    pltpu.async_copy(y.at[sidx_all.at[g * k + b]],
                                     rows[sk].at[b], gsems[sk])

            def wait_gathers(sk):
                for b in range(k):
                    pltpu.make_async_copy(y.at[pl.ds(0, CH)], rows[sk].at[b],
                                          gsems[sk]).wait()

            def start_scatters(g, sk):
                for b in range(k):
                    pltpu.async_copy(rows[sk].at[b],
                                     acc.at[didx_all.at[g * k + b]],
                                     ssems[sk], add=True)

            def wait_scatters(sk):
                for b in range(k):
                    pltpu.make_async_copy(y.at[pl.ds(0, CH)], rows[sk].at[b],
                                          ssems[sk]).wait()

            start_gathers(0, 0)
            for g in range(ngroups):
                cur = g % 2
                nxt = 1 - cur
                wait_gathers(cur)
                if g + 1 < ngroups:
                    if g >= 1:
                        wait_scatters(nxt)
                    start_gathers(g + 1, nxt)
                start_scatters(g, cur)
            wait_scatters(0)
            wait_scatters(1)

        @pl.when(c == 0)
        def _():
            run(cpw0, 0)

        @pl.when(c == 1)
        def _():
            run(cpw1, NS * cpw0)

        plsc.subcore_barrier()
        pltpu.sync_copy(acc.at[pl.ds(s * rpt, rpt)], out.at[c, pl.ds(s * rpt, rpt)])

    cpw_max = max(cpw0, cpw1)
    return pl.kernel(
        body,
        out_type=jax.ShapeDtypeStruct((NC, n_pad, d), jnp.float32),
        mesh=_mesh(),
        scratch_types=[
            pltpu.VMEM((cpw_max, CH), jnp.int32),
            pltpu.VMEM((cpw_max, CH), jnp.int32),
            pltpu.VMEM((k, CH, d), jnp.float32),
            pltpu.VMEM((k, CH, d), jnp.float32),
            pltpu.VMEM_SHARED((n_pad, d), jnp.float32),
            pltpu.SemaphoreType.DMA,
            pltpu.SemaphoreType.DMA,
            pltpu.SemaphoreType.DMA,
            pltpu.SemaphoreType.DMA,
        ],
        compiler_params=pltpu.CompilerParams(use_tc_tiling_on_sc=(d % 128 == 0)),
    )


def _make_agg_kernel_wide(n_pad, chunks, d):
    cpw0, cpw1 = _core_split(chunks, SHARE0_AGG128, 8)
    rpt = n_pad // NS

    def body(srcs, dsts, y, zeros, out, s0, s1, d0, d1, r0, r1, acc,
             gs0, gs1, ss0, ss1):
        c = lax.axis_index("c")
        s = lax.axis_index("s")
        pltpu.sync_copy(zeros.at[pl.ds(s * rpt, rpt)], acc.at[pl.ds(s * rpt, rpt)])
        plsc.subcore_barrier()

        sidx = (s0, s1)
        didx = (d0, d1)
        rows = (r0, r1)
        gsems = (gs0, gs1)
        ssems = (ss0, ss1)

        def run(cpw, start):
            base = start + s * cpw

            def load_idx(j, sk):
                pltpu.sync_copy(srcs.at[base + j], sidx[sk])
                pltpu.sync_copy(dsts.at[base + j], didx[sk])

            def start_gather(sk):
                pltpu.async_copy(y.at[sidx[sk]], rows[sk], gsems[sk])

            def wait_gather(sk):
                pltpu.make_async_copy(y.at[pl.ds(0, CH)], rows[sk],
                                      gsems[sk]).wait()

            def start_scatter(sk):
                pltpu.async_copy(rows[sk], acc.at[didx[sk]], ssems[sk],
                                 add=True)

            def wait_scatter(sk):
                pltpu.make_async_copy(y.at[pl.ds(0, CH)], rows[sk],
                                      ssems[sk]).wait()

            load_idx(0, 0)
            start_gather(0)
            for j in range(cpw):
                cur = j % 2
                nxt = 1 - cur
                if j + 1 < cpw:
                    if j >= 1:
                        wait_scatter(nxt)
                    load_idx(j + 1, nxt)
                    start_gather(nxt)
                wait_gather(cur)
                start_scatter(cur)
            wait_scatter(0)
            wait_scatter(1)

        @pl.when(c == 0)
        def _():
            run(cpw0, 0)

        @pl.when(c == 1)
        def _():
            run(cpw1, NS * cpw0)

        plsc.subcore_barrier()
        pltpu.sync_copy(acc.at[pl.ds(s * rpt, rpt)], out.at[c, pl.ds(s * rpt, rpt)])

    return pl.kernel(
        body,
        out_type=jax.ShapeDtypeStruct((NC, n_pad, d), jnp.float32),
        mesh=_mesh(),
        scratch_types=[
            pltpu.VMEM((CH,), jnp.int32),
            pltpu.VMEM((CH,), jnp.int32),
            pltpu.VMEM((CH,), jnp.int32),
            pltpu.VMEM((CH,), jnp.int32),
            pltpu.VMEM((CH, d), jnp.float32),
            pltpu.VMEM((CH, d), jnp.float32),
            pltpu.VMEM_SHARED((n_pad, d), jnp.float32),
            pltpu.SemaphoreType.DMA,
            pltpu.SemaphoreType.DMA,
            pltpu.SemaphoreType.DMA,
            pltpu.SemaphoreType.DMA,
        ],
        compiler_params=pltpu.CompilerParams(use_tc_tiling_on_sc=(d % 128 == 0)),
    )


_DOT = dict(preferred_element_type=jnp.float32, precision=lax.Precision.HIGHEST)


def _tc_pre(degout, x, w1, r):
    n, f = x.shape

    def body(deg_ref, x_ref, w_ref, y_ref, dis_ref):
        cnt = deg_ref[0, :, 0:1] + deg_ref[1, :, 0:1]
        dis = lax.rsqrt(cnt + 1.0)
        y_ref[...] = dis * jnp.dot(x_ref[...], w_ref[...], **_DOT)
        dis_ref[...] = dis

    return pl.pallas_call(
        body,
        grid=(n // r,),
        in_specs=[
            pl.BlockSpec((NC, r, 16), lambda i: (0, i, 0)),
            pl.BlockSpec((r, f), lambda i: (i, 0)),
            pl.BlockSpec((f, f), lambda i: (0, 0)),
        ],
        out_specs=[
            pl.BlockSpec((r, f), lambda i: (i, 0)),
            pl.BlockSpec((r, 1), lambda i: (i, 0)),
        ],
        out_shape=[
            jax.ShapeDtypeStruct((n, f), jnp.float32),
            jax.ShapeDtypeStruct((n, 1), jnp.float32),
        ],
    )(degout, x, w1)


def _tc_mid(agg1, y1, dis, b1, w2, r):
    n, f = y1.shape
    f2 = w2.shape[1]

    def body(a_ref, y1_ref, dis_ref, b_ref, w_ref, y2_ref):
        a = a_ref[0] + a_ref[1]
        dis = dis_ref[...]
        h = jnp.maximum(dis * (a + y1_ref[...]) + b_ref[...], 0.0)
        y2_ref[...] = dis * jnp.dot(h, w_ref[...], **_DOT)

    return pl.pallas_call(
        body,
        grid=(n // r,),
        in_specs=[
            pl.BlockSpec((NC, r, f), lambda i: (0, i, 0)),
            pl.BlockSpec((r, f), lambda i: (i, 0)),
            pl.BlockSpec((r, 1), lambda i: (i, 0)),
            pl.BlockSpec((1, f), lambda i: (0, 0)),
            pl.BlockSpec((f, f2), lambda i: (0, 0)),
        ],
        out_specs=pl.BlockSpec((r, f2), lambda i: (i, 0)),
        out_shape=jax.ShapeDtypeStruct((n, f2), jnp.float32),
    )(agg1, y1, dis, b1, w2)


def _tc_post(agg2, y2, dis, b2, fcw_pad, fcb_pad, r):
    n, f2 = y2.shape
    w = fcw_pad.shape[1]

    def body(a_ref, y2_ref, dis_ref, b_ref, fw_ref, fb_ref, o_ref):
        a = a_ref[0] + a_ref[1]
        dis = dis_ref[...]
        h = jnp.maximum(dis * (a + y2_ref[...]) + b_ref[...], 0.0)
        lp = jnp.dot(h, fw_ref[...], **_DOT) + fb_ref[...]
        m = jnp.max(lp, axis=1, keepdims=True)
        ssum = jnp.sum(jnp.exp(lp - m), axis=1, keepdims=True)
        o_ref[...] = lp - m - jnp.log(ssum)

    return pl.pallas_call(
        body,
        grid=(n // r,),
        in_specs=[
            pl.BlockSpec((NC, r, f2), lambda i: (0, i, 0)),
            pl.BlockSpec((r, f2), lambda i: (i, 0)),
            pl.BlockSpec((r, 1), lambda i: (i, 0)),
            pl.BlockSpec((1, f2), lambda i: (0, 0)),
            pl.BlockSpec((f2, w), lambda i: (0, 0)),
            pl.BlockSpec((1, w), lambda i: (0, 0)),
        ],
        out_specs=pl.BlockSpec((r, w), lambda i: (i, 0)),
        out_shape=jax.ShapeDtypeStruct((n, w), jnp.float32),
    )(agg2, y2, dis, b2, fcw_pad, fcb_pad)


def kernel(x, edge_index, W1, b1, W2, b2, fc_W, fc_b):
    n, f = x.shape
    e = edge_index.shape[1]
    f2 = W2.shape[1]
    ncls = fc_W.shape[1]
    r = 2000

    src = edge_index[0].astype(jnp.int32)
    dst = edge_index[1].astype(jnp.int32)
    cpw = -(-(-(-e // (NW * CH))) // 16) * 16
    e_pad = NW * CH * cpw
    chunks = e_pad // CH
    pad = e_pad - e
    srcs = jnp.concatenate([src, jnp.zeros((pad,), jnp.int32)]).reshape(chunks, CH)
    dsts = jnp.concatenate([dst, jnp.full((pad,), n, jnp.int32)]).reshape(chunks, CH)

    n_pad = -(-(n + 1) // (NS * 8)) * (NS * 8)
    z16 = jnp.zeros((n_pad, 16), jnp.float32)
    zf = jnp.zeros((n_pad, f), jnp.float32)
    e0 = jnp.zeros((CH, 16), jnp.float32).at[:, 0].set(1.0)

    degout = _make_deg_kernel(n_pad, chunks)(dsts, e0, z16)
    y1, dis = _tc_pre(degout, x, W1, r)
    agg1 = _make_agg_kernel_wide(n_pad, chunks, f)(srcs, dsts, y1, zf)
    y2 = _tc_mid(agg1, y1, dis, b1.reshape(1, f), W2, r)
    agg2 = _make_agg_kernel(n_pad, chunks, f2)(srcs, dsts, y2, z16[:, :f2])
    fcw_pad = jnp.zeros((f2, 128), jnp.float32).at[:, :ncls].set(fc_W)
    fcb_pad = jnp.full((1, 128), -1e30, jnp.float32).at[0, :ncls].set(fc_b)
    out = _tc_post(agg2, y2, dis, b2.reshape(1, f2), fcw_pad, fcb_pad, r)
    return out[:, :ncls]
